# baseline (speedup 1.0000x reference)
import functools

import jax
import jax.numpy as jnp
from jax import lax
from jax.experimental import pallas as pl
from jax.experimental.pallas import tpu as pltpu
from jax.experimental.pallas import tpu_sc as plsc

N = 10000
E = 320000
D = 128
NG = 512

NC = 2
NS = 16
NW = NC * NS

K = 80
CPW = E // K // NW

NZ = N // K

BN = 1000
NB = N // BN


def _sc_aggregate_body(nh_hbm, es_hbm, ed_hbm, eh_hbm, out_hbm,
                       sidx_all, didx_all, rows0, rows1, er0, er1,
                       aggr_sh, sg0, sg1, ss0, ss1):
    cid = lax.axis_index("c")
    sid = lax.axis_index("s")
    wid = sid * NC + cid

    rows = (rows0, rows1)
    er = (er0, er1)
    semg = (sg0, sg1)
    sems = (ss0, ss1)

    def base_of(c):
        return (wid * CPW + c) * K

    def issue_ge(c, b):
        pltpu.async_copy(nh_hbm.at[sidx_all.at[c]], rows[b], semg[b])
        pltpu.async_copy(eh_hbm.at[pl.ds(base_of(c), K)], er[b], semg[b])

    def wait_ge(c, b):
        pltpu.make_async_copy(nh_hbm.at[sidx_all.at[c]], rows[b],
                              semg[b]).wait()
        pltpu.make_async_copy(eh_hbm.at[pl.ds(base_of(c), K)], er[b],
                              semg[b]).wait()

    def issue_sc(c, b):
        pltpu.async_copy(er[b], aggr_sh.at[didx_all.at[c]], sems[b],
                         add=True)

    def wait_sc(c, b):
        pltpu.make_async_copy(er[b], aggr_sh.at[didx_all.at[c]],
                              sems[b]).wait()

    def compute(b):
        @plsc.parallel_loop(0, K, unroll=2)
        def _(i):
            for g in range(D // 32):
                w = rows[b][i, pl.ds(g * 16, 16)]
                xa = lax.bitcast_convert_type(lax.shift_left(w, 16),
                                              jnp.float32)
                xb = lax.bitcast_convert_type(
                    jnp.bitwise_and(w, jnp.int32(-65536)), jnp.float32)
                sla = pl.ds(g * 32, 16)
                slb = pl.ds(g * 32 + 16, 16)
                er[b][i, sla] = jnp.maximum(xa + er[b][i, sla], 0.0)
                er[b][i, slb] = jnp.maximum(xb + er[b][i, slb], 0.0)

    pltpu.sync_copy(es_hbm.at[wid], sidx_all)
    pltpu.sync_copy(ed_hbm.at[wid], didx_all)

    def _zrow(i, carry):
        for j in range(D // 16):
            er0[i, pl.ds(j * 16, 16)] = jnp.zeros((16,), jnp.float32)
        return carry
    lax.fori_loop(0, K, _zrow, 0)
    for m in range(NZ):
        @pl.when(sid == (m % NS))
        def _():
            pltpu.sync_copy(er0, aggr_sh.at[pl.ds(m * K, K)])
    plsc.subcore_barrier()

    issue_ge(0, 0)

    def group(c2, carry):
        for bb in range(2):
            c = c2 * 2 + bb
            b = bb

            @pl.when(c >= 1)
            def _():
                wait_sc(c - 1, 1 - b)

            issue_ge(c + 1, 1 - b)
            wait_ge(c, b)
            compute(b)
            issue_sc(c, b)
        return carry

    lax.fori_loop(0, (CPW - 1) // 2, group, 0)

    wait_sc(CPW - 2, 1)
    wait_ge(CPW - 1, 0)
    compute(0)
    issue_sc(CPW - 1, 0)
    wait_sc(CPW - 1, 0)

    plsc.subcore_barrier()

    for m in range(NZ):
        @pl.when(sid == (m % NS))
        def _():
            pltpu.sync_copy(aggr_sh.at[pl.ds(m * K, K)],
                            out_hbm.at[cid, pl.ds(m * K, K)])


_sc_aggregate = functools.partial(
    pl.kernel,
    out_type=jax.ShapeDtypeStruct((NC, N, D), jnp.float32),
    mesh=plsc.VectorSubcoreMesh(core_axis_name="c", subcore_axis_name="s"),
    compiler_params=pltpu.CompilerParams(use_tc_tiling_on_sc=False),
    scratch_types=[
        pltpu.VMEM((CPW, K), jnp.int32),
        pltpu.VMEM((CPW, K), jnp.int32),
        pltpu.VMEM((K, D // 2), jnp.int32),
        pltpu.VMEM((K, D // 2), jnp.int32),
        pltpu.VMEM((K, D), jnp.float32),
        pltpu.VMEM((K, D), jnp.float32),
        pltpu.VMEM_SHARED((N, D), jnp.float32),
        pltpu.SemaphoreType.DMA,
        pltpu.SemaphoreType.DMA,
        pltpu.SemaphoreType.DMA,
        pltpu.SemaphoreType.DMA,
    ],
)(_sc_aggregate_body)


def _dense_body(x_ref, p_ref, nid_ref, w1_ref, b1_ref, w2_ref,
                b2_ref, g_ref, be_ref, out_ref, cnt_s):
    ph = pl.program_id(0)
    i = pl.program_id(1)
    nid = nid_ref[0, 0, :]
    oh = (nid[:, None] == lax.broadcasted_iota(jnp.int32, (BN, NG), 1))
    ohf = oh.astype(jnp.float32)

    @pl.when(ph == 0)
    def _():
        colsum = jnp.sum(ohf, axis=0)[None, :]
        cnt_s[...] = jnp.where(i == 0, colsum, cnt_s[...] + colsum)

    @pl.when(ph == 1)
    def _():
        x = x_ref[...]
        h = x + p_ref[0] + p_ref[1]
        h = jnp.dot(h, w1_ref[...],
                    preferred_element_type=jnp.float32) + b1_ref[...]
        h = jnp.maximum(h, 0.0)
        h = jnp.dot(h, w2_ref[...],
                    preferred_element_type=jnp.float32) + b2_ref[...]
        mu = jnp.mean(h, axis=1, keepdims=True)
        xc = h - mu
        var = jnp.mean(xc * xc, axis=1, keepdims=True)
        h = xc * lax.rsqrt(var + 1e-5) * g_ref[...] + be_ref[...]
        rc = lax.rsqrt(jnp.maximum(cnt_s[...], 1.0))
        rinv = jnp.sum(ohf * rc, axis=1, keepdims=True)
        h = jnp.maximum(h * rinv, 0.0)
        out_ref[...] = h + x


def kernel(node_hidden, edge_index, edge_hidden, node_id, edge_id,
           W1, b1, W2, b2, ln_gamma, ln_beta):
    nh_bf = jnp.transpose(
        node_hidden.astype(jnp.bfloat16).reshape(N, D // 32, 2, 16),
        (0, 1, 3, 2))
    nh_words = lax.bitcast_convert_type(nh_bf, jnp.int32).reshape(N, D // 2)
    es3 = jnp.reshape(edge_index[0], (NW, CPW, K))
    ed3 = jnp.reshape(edge_index[1], (NW, CPW, K))
    partials = _sc_aggregate(nh_words, es3, ed3, edge_hidden)

    nid3 = jnp.reshape(node_id.astype(jnp.int32), (NB, 1, BN))
    out = pl.pallas_call(
        _dense_body,
        grid=(2, NB),
        in_specs=[
            pl.BlockSpec((BN, D), lambda p, i: (i * p, 0)),
            pl.BlockSpec((NC, BN, D), lambda p, i: (0, i * p, 0)),
            pl.BlockSpec((1, 1, BN), lambda p, i: (i, 0, 0)),
            pl.BlockSpec((D, 2 * D), lambda p, i: (0, 0)),
            pl.BlockSpec((1, 2 * D), lambda p, i: (0, 0)),
            pl.BlockSpec((2 * D, D), lambda p, i: (0, 0)),
            pl.BlockSpec((1, D), lambda p, i: (0, 0)),
            pl.BlockSpec((1, D), lambda p, i: (0, 0)),
            pl.BlockSpec((1, D), lambda p, i: (0, 0)),
        ],
        out_specs=pl.BlockSpec((BN, D), lambda p, i: (i * p, 0)),
        out_shape=jax.ShapeDtypeStruct((N, D), jnp.float32),
        scratch_shapes=[pltpu.VMEM((1, NG), jnp.float32)],
    )(node_hidden, partials, nid3,
      W1, jnp.reshape(b1, (1, 2 * D)), W2, jnp.reshape(b2, (1, D)),
      jnp.reshape(ln_gamma, (1, D)), jnp.reshape(ln_beta, (1, D)))
    return out


# speedup vs baseline: 1.0793x; 1.0793x over previous
import functools

import jax
import jax.numpy as jnp
from jax import lax
from jax.experimental import pallas as pl
from jax.experimental.pallas import tpu as pltpu
from jax.experimental.pallas import tpu_sc as plsc

N = 10000
E = 320000
D = 128
NG = 512

NC = 2
NS = 16
NW = NC * NS

K = 80
CPW = E // K // NW

NZ = N // K

BN = 1000
NB = N // BN


def _sc_aggregate_body(nh_hbm, ei_hbm, eh_hbm, out_hbm,
                       sidx_all, didx_all, rows0, rows1, er0, er1,
                       aggr_sh, sg0, sg1, ss0, ss1):
    cid = lax.axis_index("c")
    sid = lax.axis_index("s")
    wid = sid * NC + cid

    rows = (rows0, rows1)
    er = (er0, er1)
    semg = (sg0, sg1)
    sems = (ss0, ss1)

    def base_of(c):
        return (wid * CPW + c) * K

    def issue_ge(c, b):
        pltpu.async_copy(nh_hbm.at[sidx_all.at[c]], rows[b], semg[b])
        pltpu.async_copy(eh_hbm.at[pl.ds(base_of(c), K)], er[b], semg[b])

    def wait_ge(c, b):
        pltpu.make_async_copy(nh_hbm.at[sidx_all.at[c]], rows[b],
                              semg[b]).wait()
        pltpu.make_async_copy(eh_hbm.at[pl.ds(base_of(c), K)], er[b],
                              semg[b]).wait()

    def issue_sc(c, b):
        pltpu.async_copy(er[b], aggr_sh.at[didx_all.at[c]], sems[b],
                         add=True)

    def wait_sc(c, b):
        pltpu.make_async_copy(er[b], aggr_sh.at[didx_all.at[c]],
                              sems[b]).wait()

    def compute(b):
        @plsc.parallel_loop(0, K, unroll=2)
        def _(i):
            for g in range(D // 32):
                w = rows[b][i, pl.ds(g * 16, 16)]
                xa = lax.bitcast_convert_type(lax.shift_left(w, 16),
                                              jnp.float32)
                xb = lax.bitcast_convert_type(
                    jnp.bitwise_and(w, jnp.int32(-65536)), jnp.float32)
                sla = pl.ds(g * 32, 16)
                slb = pl.ds(g * 32 + 16, 16)
                er[b][i, sla] = jnp.maximum(xa + er[b][i, sla], 0.0)
                er[b][i, slb] = jnp.maximum(xb + er[b][i, slb], 0.0)

    pltpu.sync_copy(ei_hbm.at[0, wid], sidx_all)
    pltpu.sync_copy(ei_hbm.at[1, wid], didx_all)

    def _zrow(i, carry):
        for j in range(D // 16):
            er0[i, pl.ds(j * 16, 16)] = jnp.zeros((16,), jnp.float32)
        return carry
    lax.fori_loop(0, K, _zrow, 0)
    for m in range(NZ):
        @pl.when(sid == (m % NS))
        def _():
            pltpu.sync_copy(er0, aggr_sh.at[pl.ds(m * K, K)])
    plsc.subcore_barrier()

    issue_ge(0, 0)

    def group(c2, carry):
        for bb in range(2):
            c = c2 * 2 + bb
            b = bb

            @pl.when(c >= 1)
            def _():
                wait_sc(c - 1, 1 - b)

            issue_ge(c + 1, 1 - b)
            wait_ge(c, b)
            compute(b)
            issue_sc(c, b)
        return carry

    lax.fori_loop(0, (CPW - 1) // 2, group, 0)

    wait_sc(CPW - 2, 1)
    wait_ge(CPW - 1, 0)
    compute(0)
    issue_sc(CPW - 1, 0)
    wait_sc(CPW - 1, 0)

    plsc.subcore_barrier()

    for m in range(NZ):
        @pl.when(sid == (m % NS))
        def _():
            pltpu.sync_copy(aggr_sh.at[pl.ds(m * K, K)],
                            out_hbm.at[cid, pl.ds(m * K, K)])


_sc_aggregate = functools.partial(
    pl.kernel,
    out_type=jax.ShapeDtypeStruct((NC, N, D), jnp.float32),
    mesh=plsc.VectorSubcoreMesh(core_axis_name="c", subcore_axis_name="s"),
    compiler_params=pltpu.CompilerParams(use_tc_tiling_on_sc=False),
    scratch_types=[
        pltpu.VMEM((CPW, K), jnp.int32),
        pltpu.VMEM((CPW, K), jnp.int32),
        pltpu.VMEM((K, D // 2), jnp.int32),
        pltpu.VMEM((K, D // 2), jnp.int32),
        pltpu.VMEM((K, D), jnp.float32),
        pltpu.VMEM((K, D), jnp.float32),
        pltpu.VMEM_SHARED((N, D), jnp.float32),
        pltpu.SemaphoreType.DMA,
        pltpu.SemaphoreType.DMA,
        pltpu.SemaphoreType.DMA,
        pltpu.SemaphoreType.DMA,
    ],
)(_sc_aggregate_body)


def _counts_body(nid_ref, out_ref):
    i = pl.program_id(0)
    nid = nid_ref[0, 0, :]
    oh = (nid[:, None] == lax.broadcasted_iota(jnp.int32, (BN, NG), 1))
    colsum = jnp.sum(oh.astype(jnp.float32), axis=0)

    @pl.when(i == 0)
    def _():
        out_ref[...] = colsum[None, :]

    @pl.when(i > 0)
    def _():
        out_ref[...] = out_ref[...] + colsum[None, :]


def _dense_body(x_ref, p_ref, nid_ref, cnt_ref, w1_ref, b1_ref, w2_ref,
                b2_ref, g_ref, be_ref, out_ref):
    x = x_ref[...]
    h = x + p_ref[0] + p_ref[1]
    h = jnp.dot(h, w1_ref[...],
                preferred_element_type=jnp.float32) + b1_ref[...]
    h = jnp.maximum(h, 0.0)
    h = jnp.dot(h, w2_ref[...],
                preferred_element_type=jnp.float32) + b2_ref[...]
    mu = jnp.mean(h, axis=1, keepdims=True)
    xc = h - mu
    var = jnp.mean(xc * xc, axis=1, keepdims=True)
    h = xc * lax.rsqrt(var + 1e-5) * g_ref[...] + be_ref[...]
    nid = nid_ref[0, 0, :]
    rc = lax.rsqrt(jnp.maximum(cnt_ref[...], 1.0))
    oh = (nid[:, None] == lax.broadcasted_iota(jnp.int32, (BN, NG), 1))
    rinv = jnp.sum(oh.astype(jnp.float32) * rc, axis=1, keepdims=True)
    h = jnp.maximum(h * rinv, 0.0)
    out_ref[...] = h + x


def kernel(node_hidden, edge_index, edge_hidden, node_id, edge_id,
           W1, b1, W2, b2, ln_gamma, ln_beta):
    xb = lax.bitcast_convert_type(node_hidden, jnp.int32).reshape(N, D // 32,
                                                                  32)
    a = xb[:, :, 0:16]
    b = xb[:, :, 16:32]
    half = jnp.int32(0x8000)
    lo = lax.shift_right_logical(a + half, 16)
    hi = jnp.bitwise_and(b + half, jnp.int32(-65536))
    nh_words = jnp.bitwise_or(lo, hi).reshape(N, D // 2)

    ei4 = jnp.reshape(edge_index, (2, NW, CPW, K))
    partials = _sc_aggregate(nh_words, ei4, edge_hidden)

    nid3 = jnp.reshape(node_id.astype(jnp.int32), (NB, 1, BN))
    counts = pl.pallas_call(
        _counts_body,
        grid=(NB,),
        in_specs=[pl.BlockSpec((1, 1, BN), lambda i: (i, 0, 0))],
        out_specs=pl.BlockSpec((1, NG), lambda i: (0, 0)),
        out_shape=jax.ShapeDtypeStruct((1, NG), jnp.float32),
    )(nid3)

    out = pl.pallas_call(
        _dense_body,
        grid=(NB,),
        in_specs=[
            pl.BlockSpec((BN, D), lambda i: (i, 0)),
            pl.BlockSpec((NC, BN, D), lambda i: (0, i, 0)),
            pl.BlockSpec((1, 1, BN), lambda i: (i, 0, 0)),
            pl.BlockSpec((1, NG), lambda i: (0, 0)),
            pl.BlockSpec((D, 2 * D), lambda i: (0, 0)),
            pl.BlockSpec((1, 2 * D), lambda i: (0, 0)),
            pl.BlockSpec((2 * D, D), lambda i: (0, 0)),
            pl.BlockSpec((1, D), lambda i: (0, 0)),
            pl.BlockSpec((1, D), lambda i: (0, 0)),
            pl.BlockSpec((1, D), lambda i: (0, 0)),
        ],
        out_specs=pl.BlockSpec((BN, D), lambda i: (i, 0)),
        out_shape=jax.ShapeDtypeStruct((N, D), jnp.float32),
    )(node_hidden, partials, nid3, counts,
      W1, jnp.reshape(b1, (1, 2 * D)), W2, jnp.reshape(b2, (1, D)),
      jnp.reshape(ln_gamma, (1, D)), jnp.reshape(ln_beta, (1, D)))
    return out


# speedup vs baseline: 1.0836x; 1.0041x over previous
import functools

import jax
import jax.numpy as jnp
from jax import lax
from jax.experimental import pallas as pl
from jax.experimental.pallas import tpu as pltpu
from jax.experimental.pallas import tpu_sc as plsc

N = 10000
E = 320000
D = 128
NG = 512

NC = 2
NS = 16
NW = NC * NS

K = 80
CPW = E // K // NW

NZ = N // K

BN = 1000
NB = N // BN


def _sc_aggregate_body(nh_hbm, ei_hbm, eh_hbm, out_hbm,
                       sidx_all, didx_all, rows0, rows1, er0, er1,
                       aggr_sh, sg0, sg1, ss0, ss1):
    cid = lax.axis_index("c")
    sid = lax.axis_index("s")
    wid = sid * NC + cid

    rows = (rows0, rows1)
    er = (er0, er1)
    semg = (sg0, sg1)
    sems = (ss0, ss1)

    def base_of(c):
        return (wid * CPW + c) * K

    def issue_ge(c, b):
        pltpu.async_copy(nh_hbm.at[sidx_all.at[c]], rows[b], semg[b])
        pltpu.async_copy(eh_hbm.at[pl.ds(base_of(c), K)], er[b], semg[b])

    def wait_ge(c, b):
        pltpu.make_async_copy(nh_hbm.at[sidx_all.at[c]], rows[b],
                              semg[b]).wait()
        pltpu.make_async_copy(eh_hbm.at[pl.ds(base_of(c), K)], er[b],
                              semg[b]).wait()

    def issue_sc(c, b):
        pltpu.async_copy(er[b], aggr_sh.at[didx_all.at[c]], sems[b],
                         add=True)

    def wait_sc(c, b):
        pltpu.make_async_copy(er[b], aggr_sh.at[didx_all.at[c]],
                              sems[b]).wait()

    def compute(b):
        @plsc.parallel_loop(0, K, unroll=4)
        def _(i):
            for g in range(D // 32):
                w = rows[b][i, pl.ds(g * 16, 16)]
                xa = lax.bitcast_convert_type(lax.shift_left(w, 16),
                                              jnp.float32)
                xb = lax.bitcast_convert_type(
                    jnp.bitwise_and(w, jnp.int32(-65536)), jnp.float32)
                sla = pl.ds(g * 32, 16)
                slb = pl.ds(g * 32 + 16, 16)
                er[b][i, sla] = jnp.maximum(xa + er[b][i, sla], 0.0)
                er[b][i, slb] = jnp.maximum(xb + er[b][i, slb], 0.0)

    pltpu.sync_copy(ei_hbm.at[0, wid], sidx_all)
    pltpu.sync_copy(ei_hbm.at[1, wid], didx_all)

    def _zrow(i, carry):
        for j in range(D // 16):
            er0[i, pl.ds(j * 16, 16)] = jnp.zeros((16,), jnp.float32)
        return carry
    lax.fori_loop(0, K, _zrow, 0)
    for m in range(NZ):
        @pl.when(sid == (m % NS))
        def _():
            pltpu.sync_copy(er0, aggr_sh.at[pl.ds(m * K, K)])
    plsc.subcore_barrier()

    issue_ge(0, 0)

    def group(c2, carry):
        for bb in range(2):
            c = c2 * 2 + bb
            b = bb

            @pl.when(c >= 1)
            def _():
                wait_sc(c - 1, 1 - b)

            issue_ge(c + 1, 1 - b)
            wait_ge(c, b)
            compute(b)
            issue_sc(c, b)
        return carry

    lax.fori_loop(0, (CPW - 1) // 2, group, 0)

    wait_sc(CPW - 2, 1)
    wait_ge(CPW - 1, 0)
    compute(0)
    issue_sc(CPW - 1, 0)
    wait_sc(CPW - 1, 0)

    plsc.subcore_barrier()

    for m in range(NZ):
        @pl.when(sid == (m % NS))
        def _():
            pltpu.sync_copy(aggr_sh.at[pl.ds(m * K, K)],
                            out_hbm.at[cid, pl.ds(m * K, K)])


_sc_aggregate = functools.partial(
    pl.kernel,
    out_type=jax.ShapeDtypeStruct((NC, N, D), jnp.float32),
    mesh=plsc.VectorSubcoreMesh(core_axis_name="c", subcore_axis_name="s"),
    compiler_params=pltpu.CompilerParams(use_tc_tiling_on_sc=False),
    scratch_types=[
        pltpu.VMEM((CPW, K), jnp.int32),
        pltpu.VMEM((CPW, K), jnp.int32),
        pltpu.VMEM((K, D // 2), jnp.int32),
        pltpu.VMEM((K, D // 2), jnp.int32),
        pltpu.VMEM((K, D), jnp.float32),
        pltpu.VMEM((K, D), jnp.float32),
        pltpu.VMEM_SHARED((N, D), jnp.float32),
        pltpu.SemaphoreType.DMA,
        pltpu.SemaphoreType.DMA,
        pltpu.SemaphoreType.DMA,
        pltpu.SemaphoreType.DMA,
    ],
)(_sc_aggregate_body)


def _counts_body(nid_ref, out_ref):
    i = pl.program_id(0)
    nid = nid_ref[0, 0, :]
    oh = (nid[:, None] == lax.broadcasted_iota(jnp.int32, (BN, NG), 1))
    colsum = jnp.sum(oh.astype(jnp.float32), axis=0)

    @pl.when(i == 0)
    def _():
        out_ref[...] = colsum[None, :]

    @pl.when(i > 0)
    def _():
        out_ref[...] = out_ref[...] + colsum[None, :]


def _dense_body(x_ref, p_ref, nid_ref, cnt_ref, w1_ref, b1_ref, w2_ref,
                b2_ref, g_ref, be_ref, out_ref):
    x = x_ref[...]
    h = x + p_ref[0] + p_ref[1]
    h = jnp.dot(h, w1_ref[...],
                preferred_element_type=jnp.float32) + b1_ref[...]
    h = jnp.maximum(h, 0.0)
    h = jnp.dot(h, w2_ref[...],
                preferred_element_type=jnp.float32) + b2_ref[...]
    mu = jnp.mean(h, axis=1, keepdims=True)
    xc = h - mu
    var = jnp.mean(xc * xc, axis=1, keepdims=True)
    h = xc * lax.rsqrt(var + 1e-5) * g_ref[...] + be_ref[...]
    nid = nid_ref[0, 0, :]
    rc = lax.rsqrt(jnp.maximum(cnt_ref[...], 1.0))
    oh = (nid[:, None] == lax.broadcasted_iota(jnp.int32, (BN, NG), 1))
    rinv = jnp.sum(oh.astype(jnp.float32) * rc, axis=1, keepdims=True)
    h = jnp.maximum(h * rinv, 0.0)
    out_ref[...] = h + x


def kernel(node_hidden, edge_index, edge_hidden, node_id, edge_id,
           W1, b1, W2, b2, ln_gamma, ln_beta):
    xb = lax.bitcast_convert_type(node_hidden, jnp.int32).reshape(N, D // 32,
                                                                  32)
    a = xb[:, :, 0:16]
    b = xb[:, :, 16:32]
    half = jnp.int32(0x8000)
    lo = lax.shift_right_logical(a + half, 16)
    hi = jnp.bitwise_and(b + half, jnp.int32(-65536))
    nh_words = jnp.bitwise_or(lo, hi).reshape(N, D // 2)

    ei4 = jnp.reshape(edge_index, (2, NW, CPW, K))
    partials = _sc_aggregate(nh_words, ei4, edge_hidden)

    nid3 = jnp.reshape(node_id.astype(jnp.int32), (NB, 1, BN))
    counts = pl.pallas_call(
        _counts_body,
        grid=(NB,),
        in_specs=[pl.BlockSpec((1, 1, BN), lambda i: (i, 0, 0))],
        out_specs=pl.BlockSpec((1, NG), lambda i: (0, 0)),
        out_shape=jax.ShapeDtypeStruct((1, NG), jnp.float32),
    )(nid3)

    out = pl.pallas_call(
        _dense_body,
        grid=(NB,),
        in_specs=[
            pl.BlockSpec((BN, D), lambda i: (i, 0)),
            pl.BlockSpec((NC, BN, D), lambda i: (0, i, 0)),
            pl.BlockSpec((1, 1, BN), lambda i: (i, 0, 0)),
            pl.BlockSpec((1, NG), lambda i: (0, 0)),
            pl.BlockSpec((D, 2 * D), lambda i: (0, 0)),
            pl.BlockSpec((1, 2 * D), lambda i: (0, 0)),
            pl.BlockSpec((2 * D, D), lambda i: (0, 0)),
            pl.BlockSpec((1, D), lambda i: (0, 0)),
            pl.BlockSpec((1, D), lambda i: (0, 0)),
            pl.BlockSpec((1, D), lambda i: (0, 0)),
        ],
        out_specs=pl.BlockSpec((BN, D), lambda i: (i, 0)),
        out_shape=jax.ShapeDtypeStruct((N, D), jnp.float32),
    )(node_hidden, partials, nid3, counts,
      W1, jnp.reshape(b1, (1, 2 * D)), W2, jnp.reshape(b2, (1, D)),
      jnp.reshape(ln_gamma, (1, D)), jnp.reshape(ln_beta, (1, D)))
    return out


# speedup vs baseline: 1.0882x; 1.0042x over previous
import functools

import jax
import jax.numpy as jnp
from jax import lax
from jax.experimental import pallas as pl
from jax.experimental.pallas import tpu as pltpu
from jax.experimental.pallas import tpu_sc as plsc

N = 10000
E = 320000
D = 128
NG = 512

NC = 2
NS = 16
NW = NC * NS

K = 80
CPW = E // K // NW

NZ = N // K

BN = 1000
NB = N // BN


def _sc_aggregate_body(nh_hbm, ei_hbm, eh_hbm, out_hbm,
                       sidx_all, didx_all, rows0, rows1, er0, er1,
                       aggr_sh, sg0, sg1, ss0, ss1):
    cid = lax.axis_index("c")
    sid = lax.axis_index("s")
    wid = sid * NC + cid

    rows = (rows0, rows1)
    er = (er0, er1)
    semg = (sg0, sg1)
    sems = (ss0, ss1)

    def base_of(c):
        return (wid * CPW + c) * K

    def issue_ge(c, b):
        pltpu.async_copy(nh_hbm.at[sidx_all.at[c]], rows[b], semg[b])
        pltpu.async_copy(eh_hbm.at[pl.ds(base_of(c), K)], er[b], semg[b])

    def wait_ge(c, b):
        pltpu.make_async_copy(nh_hbm.at[sidx_all.at[c]], rows[b],
                              semg[b]).wait()
        pltpu.make_async_copy(eh_hbm.at[pl.ds(base_of(c), K)], er[b],
                              semg[b]).wait()

    def issue_sc(c, b):
        pltpu.async_copy(er[b], aggr_sh.at[didx_all.at[c]], sems[b],
                         add=True)

    def wait_sc(c, b):
        pltpu.make_async_copy(er[b], aggr_sh.at[didx_all.at[c]],
                              sems[b]).wait()

    def compute(b):
        @plsc.parallel_loop(0, K, unroll=4)
        def _(i):
            for g in range(D // 32):
                w = rows[b][i, pl.ds(g * 16, 16)]
                xa = lax.bitcast_convert_type(lax.shift_left(w, 16),
                                              jnp.float32)
                xb = lax.bitcast_convert_type(
                    jnp.bitwise_and(w, jnp.int32(-65536)), jnp.float32)
                sla = pl.ds(g * 32, 16)
                slb = pl.ds(g * 32 + 16, 16)
                er[b][i, sla] = jnp.maximum(xa + er[b][i, sla], 0.0)
                er[b][i, slb] = jnp.maximum(xb + er[b][i, slb], 0.0)

    pltpu.async_copy(ei_hbm.at[0, wid], sidx_all, sg0)
    pltpu.async_copy(ei_hbm.at[1, wid], didx_all, sg1)

    def _zrow(i, carry):
        for j in range(D // 16):
            er0[i, pl.ds(j * 16, 16)] = jnp.zeros((16,), jnp.float32)
        return carry
    lax.fori_loop(0, K, _zrow, 0)
    pltpu.make_async_copy(ei_hbm.at[0, wid], sidx_all, sg0).wait()
    pltpu.make_async_copy(ei_hbm.at[1, wid], didx_all, sg1).wait()
    for m in range(NZ):
        @pl.when(sid == (m % NS))
        def _():
            pltpu.sync_copy(er0, aggr_sh.at[pl.ds(m * K, K)])
    plsc.subcore_barrier()

    issue_ge(0, 0)

    def group(c2, carry):
        for bb in range(2):
            c = c2 * 2 + bb
            b = bb

            @pl.when(c >= 1)
            def _():
                wait_sc(c - 1, 1 - b)

            issue_ge(c + 1, 1 - b)
            wait_ge(c, b)
            compute(b)
            issue_sc(c, b)
        return carry

    lax.fori_loop(0, (CPW - 1) // 2, group, 0)

    wait_sc(CPW - 2, 1)
    wait_ge(CPW - 1, 0)
    compute(0)
    issue_sc(CPW - 1, 0)
    wait_sc(CPW - 1, 0)

    plsc.subcore_barrier()

    for m in range(NZ):
        @pl.when(sid == (m % NS))
        def _():
            pltpu.sync_copy(aggr_sh.at[pl.ds(m * K, K)],
                            out_hbm.at[cid, pl.ds(m * K, K)])


_sc_aggregate = functools.partial(
    pl.kernel,
    out_type=jax.ShapeDtypeStruct((NC, N, D), jnp.float32),
    mesh=plsc.VectorSubcoreMesh(core_axis_name="c", subcore_axis_name="s"),
    compiler_params=pltpu.CompilerParams(use_tc_tiling_on_sc=False),
    scratch_types=[
        pltpu.VMEM((CPW, K), jnp.int32),
        pltpu.VMEM((CPW, K), jnp.int32),
        pltpu.VMEM((K, D // 2), jnp.int32),
        pltpu.VMEM((K, D // 2), jnp.int32),
        pltpu.VMEM((K, D), jnp.float32),
        pltpu.VMEM((K, D), jnp.float32),
        pltpu.VMEM_SHARED((N, D), jnp.float32),
        pltpu.SemaphoreType.DMA,
        pltpu.SemaphoreType.DMA,
        pltpu.SemaphoreType.DMA,
        pltpu.SemaphoreType.DMA,
    ],
)(_sc_aggregate_body)


def _counts_body(nid_ref, out_ref):
    i = pl.program_id(0)
    nid = nid_ref[0, 0, :]
    oh = (nid[:, None] == lax.broadcasted_iota(jnp.int32, (BN, NG), 1))
    colsum = jnp.sum(oh.astype(jnp.float32), axis=0)

    @pl.when(i == 0)
    def _():
        out_ref[...] = colsum[None, :]

    @pl.when(i > 0)
    def _():
        out_ref[...] = out_ref[...] + colsum[None, :]


def _dense_body(x_ref, p_ref, nid_ref, cnt_ref, w1_ref, b1_ref, w2_ref,
                b2_ref, g_ref, be_ref, out_ref):
    x = x_ref[...]
    h = x + p_ref[0] + p_ref[1]
    h = jnp.dot(h, w1_ref[...],
                preferred_element_type=jnp.float32) + b1_ref[...]
    h = jnp.maximum(h, 0.0)
    h = jnp.dot(h, w2_ref[...],
                preferred_element_type=jnp.float32) + b2_ref[...]
    mu = jnp.mean(h, axis=1, keepdims=True)
    xc = h - mu
    var = jnp.mean(xc * xc, axis=1, keepdims=True)
    h = xc * lax.rsqrt(var + 1e-5) * g_ref[...] + be_ref[...]
    nid = nid_ref[0, 0, :]
    rc = lax.rsqrt(jnp.maximum(cnt_ref[...], 1.0))
    oh = (nid[:, None] == lax.broadcasted_iota(jnp.int32, (BN, NG), 1))
    rinv = jnp.sum(oh.astype(jnp.float32) * rc, axis=1, keepdims=True)
    h = jnp.maximum(h * rinv, 0.0)
    out_ref[...] = h + x


def kernel(node_hidden, edge_index, edge_hidden, node_id, edge_id,
           W1, b1, W2, b2, ln_gamma, ln_beta):
    xb = lax.bitcast_convert_type(node_hidden, jnp.int32).reshape(N, D // 32,
                                                                  32)
    a = xb[:, :, 0:16]
    b = xb[:, :, 16:32]
    half = jnp.int32(0x8000)
    lo = lax.shift_right_logical(a + half, 16)
    hi = jnp.bitwise_and(b + half, jnp.int32(-65536))
    nh_words = jnp.bitwise_or(lo, hi).reshape(N, D // 2)

    ei4 = jnp.reshape(edge_index, (2, NW, CPW, K))
    partials = _sc_aggregate(nh_words, ei4, edge_hidden)

    nid3 = jnp.reshape(node_id.astype(jnp.int32), (NB, 1, BN))
    counts = pl.pallas_call(
        _counts_body,
        grid=(NB,),
        in_specs=[pl.BlockSpec((1, 1, BN), lambda i: (i, 0, 0))],
        out_specs=pl.BlockSpec((1, NG), lambda i: (0, 0)),
        out_shape=jax.ShapeDtypeStruct((1, NG), jnp.float32),
    )(nid3)

    out = pl.pallas_call(
        _dense_body,
        grid=(NB,),
        in_specs=[
            pl.BlockSpec((BN, D), lambda i: (i, 0)),
            pl.BlockSpec((NC, BN, D), lambda i: (0, i, 0)),
            pl.BlockSpec((1, 1, BN), lambda i: (i, 0, 0)),
            pl.BlockSpec((1, NG), lambda i: (0, 0)),
            pl.BlockSpec((D, 2 * D), lambda i: (0, 0)),
            pl.BlockSpec((1, 2 * D), lambda i: (0, 0)),
            pl.BlockSpec((2 * D, D), lambda i: (0, 0)),
            pl.BlockSpec((1, D), lambda i: (0, 0)),
            pl.BlockSpec((1, D), lambda i: (0, 0)),
            pl.BlockSpec((1, D), lambda i: (0, 0)),
        ],
        out_specs=pl.BlockSpec((BN, D), lambda i: (i, 0)),
        out_shape=jax.ShapeDtypeStruct((N, D), jnp.float32),
    )(node_hidden, partials, nid3, counts,
      W1, jnp.reshape(b1, (1, 2 * D)), W2, jnp.reshape(b2, (1, D)),
      jnp.reshape(ln_gamma, (1, D)), jnp.reshape(ln_beta, (1, D)))
    return out


# speedup vs baseline: 1.0966x; 1.0077x over previous
import functools

import jax
import jax.numpy as jnp
from jax import lax
from jax.experimental import pallas as pl
from jax.experimental.pallas import tpu as pltpu
from jax.experimental.pallas import tpu_sc as plsc

N = 10000
E = 320000
D = 128
NG = 512

NC = 2
NS = 16
NW = NC * NS

K = 80
CPW = E // K // NW

NZ = N // K

BN = 2000
NB = N // BN


def _sc_aggregate_body(nh_hbm, ei_hbm, eh_hbm, out_hbm,
                       sidx_all, didx_all, rows0, rows1, er0, er1,
                       aggr_sh, sg0, sg1, ss0, ss1):
    cid = lax.axis_index("c")
    sid = lax.axis_index("s")
    wid = sid * NC + cid

    rows = (rows0, rows1)
    er = (er0, er1)
    semg = (sg0, sg1)
    sems = (ss0, ss1)

    def base_of(c):
        return (wid * CPW + c) * K

    def issue_ge(c, b):
        pltpu.async_copy(nh_hbm.at[sidx_all.at[c]], rows[b], semg[b])
        pltpu.async_copy(eh_hbm.at[pl.ds(base_of(c), K)], er[b], semg[b])

    def wait_ge(c, b):
        pltpu.make_async_copy(nh_hbm.at[sidx_all.at[c]], rows[b],
                              semg[b]).wait()
        pltpu.make_async_copy(eh_hbm.at[pl.ds(base_of(c), K)], er[b],
                              semg[b]).wait()

    def issue_sc(c, b):
        pltpu.async_copy(er[b], aggr_sh.at[didx_all.at[c]], sems[b],
                         add=True)

    def wait_sc(c, b):
        pltpu.make_async_copy(er[b], aggr_sh.at[didx_all.at[c]],
                              sems[b]).wait()

    def compute(b):
        @plsc.parallel_loop(0, K, unroll=4)
        def _(i):
            for g in range(D // 32):
                w = rows[b][i, pl.ds(g * 16, 16)]
                xa = lax.bitcast_convert_type(lax.shift_left(w, 16),
                                              jnp.float32)
                xb = lax.bitcast_convert_type(
                    jnp.bitwise_and(w, jnp.int32(-65536)), jnp.float32)
                sla = pl.ds(g * 32, 16)
                slb = pl.ds(g * 32 + 16, 16)
                er[b][i, sla] = jnp.maximum(xa + er[b][i, sla], 0.0)
                er[b][i, slb] = jnp.maximum(xb + er[b][i, slb], 0.0)

    pltpu.async_copy(ei_hbm.at[0, wid], sidx_all, sg0)
    pltpu.async_copy(ei_hbm.at[1, wid], didx_all, sg1)

    def _zrow(i, carry):
        for j in range(D // 16):
            er0[i, pl.ds(j * 16, 16)] = jnp.zeros((16,), jnp.float32)
        return carry
    lax.fori_loop(0, K, _zrow, 0)
    pltpu.make_async_copy(ei_hbm.at[0, wid], sidx_all, sg0).wait()
    pltpu.make_async_copy(ei_hbm.at[1, wid], didx_all, sg1).wait()
    for m in range(NZ):
        @pl.when(sid == (m % NS))
        def _():
            pltpu.sync_copy(er0, aggr_sh.at[pl.ds(m * K, K)])
    plsc.subcore_barrier()

    issue_ge(0, 0)

    def group(c2, carry):
        for bb in range(2):
            c = c2 * 2 + bb
            b = bb

            @pl.when(c >= 1)
            def _():
                wait_sc(c - 1, 1 - b)

            issue_ge(c + 1, 1 - b)
            wait_ge(c, b)
            compute(b)
            issue_sc(c, b)
        return carry

    lax.fori_loop(0, (CPW - 1) // 2, group, 0)

    wait_sc(CPW - 2, 1)
    wait_ge(CPW - 1, 0)
    compute(0)
    issue_sc(CPW - 1, 0)
    wait_sc(CPW - 1, 0)

    plsc.subcore_barrier()

    for m in range(NZ):
        @pl.when(sid == (m % NS))
        def _():
            pltpu.sync_copy(aggr_sh.at[pl.ds(m * K, K)],
                            out_hbm.at[cid, pl.ds(m * K, K)])


_sc_aggregate = functools.partial(
    pl.kernel,
    out_type=jax.ShapeDtypeStruct((NC, N, D), jnp.float32),
    mesh=plsc.VectorSubcoreMesh(core_axis_name="c", subcore_axis_name="s"),
    compiler_params=pltpu.CompilerParams(use_tc_tiling_on_sc=False),
    scratch_types=[
        pltpu.VMEM((CPW, K), jnp.int32),
        pltpu.VMEM((CPW, K), jnp.int32),
        pltpu.VMEM((K, D // 2), jnp.int32),
        pltpu.VMEM((K, D // 2), jnp.int32),
        pltpu.VMEM((K, D), jnp.float32),
        pltpu.VMEM((K, D), jnp.float32),
        pltpu.VMEM_SHARED((N, D), jnp.float32),
        pltpu.SemaphoreType.DMA,
        pltpu.SemaphoreType.DMA,
        pltpu.SemaphoreType.DMA,
        pltpu.SemaphoreType.DMA,
    ],
)(_sc_aggregate_body)


def _counts_body(nid_ref, out_ref):
    i = pl.program_id(0)
    nid = nid_ref[0, 0, :]
    oh = (nid[:, None] == lax.broadcasted_iota(jnp.int32, (BN, NG), 1))
    colsum = jnp.sum(oh.astype(jnp.float32), axis=0)

    @pl.when(i == 0)
    def _():
        out_ref[...] = colsum[None, :]

    @pl.when(i > 0)
    def _():
        out_ref[...] = out_ref[...] + colsum[None, :]


def _dense_body(x_ref, p_ref, nid_ref, cnt_ref, w1_ref, b1_ref, w2_ref,
                b2_ref, g_ref, be_ref, out_ref):
    x = x_ref[...]
    h = x + p_ref[0] + p_ref[1]
    h = jnp.dot(h, w1_ref[...],
                preferred_element_type=jnp.float32) + b1_ref[...]
    h = jnp.maximum(h, 0.0)
    h = jnp.dot(h, w2_ref[...],
                preferred_element_type=jnp.float32) + b2_ref[...]
    mu = jnp.mean(h, axis=1, keepdims=True)
    xc = h - mu
    var = jnp.mean(xc * xc, axis=1, keepdims=True)
    h = xc * lax.rsqrt(var + 1e-5) * g_ref[...] + be_ref[...]
    nid = nid_ref[0, 0, :]
    rc = lax.rsqrt(jnp.maximum(cnt_ref[...], 1.0))
    oh = (nid[:, None] == lax.broadcasted_iota(jnp.int32, (BN, NG), 1))
    rinv = jnp.sum(oh.astype(jnp.float32) * rc, axis=1, keepdims=True)
    h = jnp.maximum(h * rinv, 0.0)
    out_ref[...] = h + x


def kernel(node_hidden, edge_index, edge_hidden, node_id, edge_id,
           W1, b1, W2, b2, ln_gamma, ln_beta):
    xb = lax.bitcast_convert_type(node_hidden, jnp.int32).reshape(N, D // 32,
                                                                  32)
    a = xb[:, :, 0:16]
    b = xb[:, :, 16:32]
    half = jnp.int32(0x8000)
    lo = lax.shift_right_logical(a + half, 16)
    hi = jnp.bitwise_and(b + half, jnp.int32(-65536))
    nh_words = jnp.bitwise_or(lo, hi).reshape(N, D // 2)

    ei4 = jnp.reshape(edge_index, (2, NW, CPW, K))
    partials = _sc_aggregate(nh_words, ei4, edge_hidden)

    nid3 = jnp.reshape(node_id.astype(jnp.int32), (NB, 1, BN))
    counts = pl.pallas_call(
        _counts_body,
        grid=(NB,),
        in_specs=[pl.BlockSpec((1, 1, BN), lambda i: (i, 0, 0))],
        out_specs=pl.BlockSpec((1, NG), lambda i: (0, 0)),
        out_shape=jax.ShapeDtypeStruct((1, NG), jnp.float32),
    )(nid3)

    out = pl.pallas_call(
        _dense_body,
        grid=(NB,),
        in_specs=[
            pl.BlockSpec((BN, D), lambda i: (i, 0)),
            pl.BlockSpec((NC, BN, D), lambda i: (0, i, 0)),
            pl.BlockSpec((1, 1, BN), lambda i: (i, 0, 0)),
            pl.BlockSpec((1, NG), lambda i: (0, 0)),
            pl.BlockSpec((D, 2 * D), lambda i: (0, 0)),
            pl.BlockSpec((1, 2 * D), lambda i: (0, 0)),
            pl.BlockSpec((2 * D, D), lambda i: (0, 0)),
            pl.BlockSpec((1, D), lambda i: (0, 0)),
            pl.BlockSpec((1, D), lambda i: (0, 0)),
            pl.BlockSpec((1, D), lambda i: (0, 0)),
        ],
        out_specs=pl.BlockSpec((BN, D), lambda i: (i, 0)),
        out_shape=jax.ShapeDtypeStruct((N, D), jnp.float32),
    )(node_hidden, partials, nid3, counts,
      W1, jnp.reshape(b1, (1, 2 * D)), W2, jnp.reshape(b2, (1, D)),
      jnp.reshape(ln_gamma, (1, D)), jnp.reshape(ln_beta, (1, D)))
    return out
